# baseline (speedup 1.0000x reference)
import functools

import jax
import jax.numpy as jnp
from jax import lax
from jax.experimental import pallas as pl
from jax.experimental.pallas import tpu as pltpu
from jax.experimental.pallas import tpu_sc as plsc

DIM = 64
LANES = 16
IDX_W = 128
NUM_CORES = 2
NUM_SUBCORES = 16
NUM_WORKERS = NUM_CORES * NUM_SUBCORES

_GDN = lax.GatherDimensionNumbers(
    offset_dims=(), collapsed_slice_dims=(0,), start_index_map=(0,))


def _xlane_take(x, perm):
    return lax.gather(x, perm[:, None], dimension_numbers=_GDN,
                      slice_sizes=(1,),
                      mode=lax.GatherScatterMode.PROMISE_IN_BOUNDS)


def _mf_bias_call(u2d, i2d, user_factors, item_factors, ub, ib, gb16, batch):
    chunk = batch // NUM_WORKERS
    nblk = chunk // IDX_W
    ngrp = chunk // LANES

    mesh = plsc.VectorSubcoreMesh(core_axis_name="c", subcore_axis_name="s")

    @functools.partial(
        pl.kernel,
        mesh=mesh,
        compiler_params=pltpu.CompilerParams(use_tc_tiling_on_sc=False),
        out_type=jax.ShapeDtypeStruct((batch,), jnp.float32),
        scratch_types=[
            pltpu.VMEM((nblk, IDX_W), jnp.int32),
            pltpu.VMEM((nblk, IDX_W), jnp.int32),
            pltpu.VMEM((chunk, DIM), jnp.float32),
            pltpu.VMEM((chunk, DIM), jnp.float32),
            pltpu.VMEM((chunk,), jnp.float32),
            pltpu.VMEM((chunk,), jnp.float32),
            pltpu.VMEM((LANES,), jnp.float32),
            pltpu.VMEM((chunk,), jnp.float32),
            pltpu.SemaphoreType.DMA,
        ],
    )
    def mf_kernel(u_hbm, i_hbm, uf_hbm, if_hbm, ub_hbm, ib_hbm, gb_hbm,
                  out_hbm, u_v, i_v, p_v, q_v, bu_v, bi_v, gb_v, o_v, sem):
        wid = lax.axis_index("s") * NUM_CORES + lax.axis_index("c")
        base = wid * chunk

        pltpu.sync_copy(u_hbm.at[pl.ds(wid * nblk, nblk)], u_v)
        pltpu.sync_copy(i_hbm.at[pl.ds(wid * nblk, nblk)], i_v)
        pltpu.sync_copy(gb_hbm, gb_v)

        copies = []
        for j in range(nblk):
            copies.append(pltpu.async_copy(
                uf_hbm.at[u_v.at[j]], p_v.at[pl.ds(j * IDX_W, IDX_W)], sem))
            copies.append(pltpu.async_copy(
                if_hbm.at[i_v.at[j]], q_v.at[pl.ds(j * IDX_W, IDX_W)], sem))
            copies.append(pltpu.async_copy(
                ub_hbm.at[u_v.at[j]], bu_v.at[pl.ds(j * IDX_W, IDX_W)], sem))
            copies.append(pltpu.async_copy(
                ib_hbm.at[i_v.at[j]], bi_v.at[pl.ds(j * IDX_W, IDX_W)], sem))
        for c in copies:
            c.wait()

        gb = gb_v[...]
        lanes = lax.iota(jnp.int32, LANES)
        perms = [jnp.bitwise_xor(lanes, sh) for sh in (8, 4, 2, 1)]

        def group_body(g, _):
            dot = jnp.zeros((LANES,), jnp.float32)
            for l in range(LANES):
                row = g * LANES + l
                acc = p_v[row, pl.ds(0, LANES)] * q_v[row, pl.ds(0, LANES)]
                for j in range(1, DIM // LANES):
                    acc = acc + (p_v[row, pl.ds(j * LANES, LANES)]
                                 * q_v[row, pl.ds(j * LANES, LANES)])
                for perm in perms:
                    acc = acc + _xlane_take(acc, perm)
                dot = jnp.where(lanes == l, acc, dot)
            sl = pl.ds(g * LANES, LANES)
            o_v[sl] = dot + bu_v[sl] + bi_v[sl] + gb
            return _

        lax.fori_loop(0, ngrp, group_body, None)
        pltpu.sync_copy(o_v, out_hbm.at[pl.ds(base, chunk)])

    return mf_kernel(u2d, i2d, user_factors, item_factors, ub, ib, gb16)


def kernel(u, i, user_factors, item_factors, user_bias, item_bias,
           global_bias):
    batch = u.shape[0]
    u2d = u.reshape(batch // IDX_W, IDX_W)
    i2d = i.reshape(batch // IDX_W, IDX_W)
    ub = user_bias.reshape(-1)
    ib = item_bias.reshape(-1)
    gb16 = jnp.broadcast_to(global_bias.astype(jnp.float32), (LANES,))
    return _mf_bias_call(u2d, i2d, user_factors, item_factors, ub, ib, gb16,
                         batch)
